# baseline (speedup 1.0000x reference)
import functools

import jax
import jax.numpy as jnp
from jax import lax
from jax.experimental import pallas as pl
from jax.experimental.pallas import tpu as pltpu
from jax.experimental.pallas import tpu_sc as plsc

_TOKENS = 32768
_HIDDEN = 768
_E = 64
_K = 8
_BT = 4096

_NW = 32
_NTOK = _TOKENS // _NW
_CH = 256


def _matmul_body(x_ref, wt_ref, logits_ref):
    logits_ref[...] = jax.lax.dot_general(
        x_ref[...], wt_ref[...], (((1,), (0,)), ((), ())),
        preferred_element_type=jnp.float32,
    )


def _tc_logits(hidden_states, wt):
    return pl.pallas_call(
        _matmul_body,
        grid=(_TOKENS // _BT,),
        in_specs=[
            pl.BlockSpec((_BT, _HIDDEN), lambda i: (i, 0)),
            pl.BlockSpec((_HIDDEN, _E), lambda i: (0, 0)),
        ],
        out_specs=pl.BlockSpec((_BT, _E), lambda i: (i, 0)),
        out_shape=jax.ShapeDtypeStruct((_TOKENS, _E), jnp.float32),
        compiler_params=pltpu.CompilerParams(
            dimension_semantics=("arbitrary",),
        ),
    )(hidden_states, wt)


def _merge_top(ak, av, bk, bv):
    rbk = lax.rev(bk, (0,))
    rbv = lax.rev(bv, (0,))
    take_a = ak >= rbk
    mk = jnp.where(take_a, ak, rbk)
    mv = jnp.where(take_a, av, rbv)
    return plsc.sort_key_val(mk, mv, descending=True)


def _sc_route_kernel(logits_hbm, out_hbm, chunk, outbuf):
    nc = 2
    wid = lax.axis_index("s") * nc + lax.axis_index("c")
    base = wid * _NTOK

    lane = lax.iota(jnp.int32, 16)
    lo8 = lane < 8

    def body(t, _):
        ks, vs = [], []
        for j in range(4):
            k = chunk[t, pl.ds(j * 16, 16)]
            v = lane + jnp.int32(j * 16)
            sk, sv = plsc.sort_key_val(k, v, descending=True)
            ks.append(sk)
            vs.append(sv)
        k01, v01 = _merge_top(ks[0], vs[0], ks[1], vs[1])
        k23, v23 = _merge_top(ks[2], vs[2], ks[3], vs[3])
        kf, vf = _merge_top(k01, v01, k23, v23)

        top = jnp.max(kf, axis=0)
        e = jnp.exp(kf - top)
        e = jnp.where(lo8, e, jnp.float32(0.0))
        w = e / jnp.sum(e, axis=0)

        vff = vf.astype(jnp.float32)
        comb = jnp.where(lo8, w, lax.rev(vff, (0,)))
        outbuf[t, :] = comb
        return _

    for c in range(_NTOK // _CH):
        cbase = base + c * _CH
        pltpu.sync_copy(logits_hbm.at[pl.ds(cbase, _CH)], chunk)
        lax.fori_loop(0, _CH, body, 0)
        pltpu.sync_copy(outbuf, out_hbm.at[pl.ds(cbase, _CH)])


def _sc_route(logits):
    mesh = plsc.VectorSubcoreMesh(core_axis_name="c", subcore_axis_name="s")
    f = functools.partial(
        pl.kernel,
        out_type=jax.ShapeDtypeStruct((_TOKENS, 16), jnp.float32),
        mesh=mesh,
        scratch_types=[
            pltpu.VMEM((_CH, _E), jnp.float32),
            pltpu.VMEM((_CH, 16), jnp.float32),
        ],
        compiler_params=pltpu.CompilerParams(needs_layout_passes=False),
    )(_sc_route_kernel)
    return f(logits)


@jax.jit
def kernel(hidden_states, weight):
    wt = weight.T
    logits = _tc_logits(hidden_states, wt)
    comb = _sc_route(logits)
    topw = comb[:, :_K]
    topi = comb[:, 15:7:-1].astype(jnp.int32)
    return (logits, topw, topi)


# speedup vs baseline: 4.7043x; 4.7043x over previous
import jax
import jax.numpy as jnp
from jax.experimental import pallas as pl
from jax.experimental.pallas import tpu as pltpu

_TOKENS = 32768
_HIDDEN = 768
_E = 64
_K = 8
_BT = 4096


def _router_body(x_ref, wt_ref, logits_ref, wi_ref):
    x = x_ref[...]
    wt = wt_ref[...]
    logits = jax.lax.dot_general(
        x, wt, (((1,), (0,)), ((), ())),
        preferred_element_type=jnp.float32,
    )
    logits_ref[...] = logits

    lt = logits.T

    ikey = lt.view(jnp.int32)
    ikey = jnp.bitwise_xor(ikey, jnp.right_shift(ikey, 31) & jnp.int32(0x7FFFFFFF))
    eidx = jax.lax.broadcasted_iota(jnp.int32, (_E, _BT), 0)

    sel_v, sel_i = [], []
    cur = ikey
    neg_inf = jnp.int32(-0x80000000)
    for _ in range(_K):
        ck, ci = cur, eidx
        r = _E
        while r > 1:
            r //= 2
            ak, bk = ck[:r], ck[r:]
            ai, bi = ci[:r], ci[r:]
            take = ak >= bk
            ck = jnp.where(take, ak, bk)
            ci = jnp.where(take, ai, bi)
        m, ix = ck, ci
        sel_v.append(m)
        sel_i.append(ix)
        cur = jnp.where(eidx == ix, neg_inf, cur)
    vk = jnp.concatenate(sel_v, axis=0)
    idx = jnp.concatenate(sel_i, axis=0)

    vals = jnp.bitwise_xor(
        vk, jnp.right_shift(vk, 31) & jnp.int32(0x7FFFFFFF)
    ).view(jnp.float32)

    e = jnp.exp(vals - vals[0:1])
    w = e / jnp.sum(e, axis=0, keepdims=True)

    wi_ref[...] = jnp.concatenate([w, idx.astype(jnp.float32)], axis=0)


@jax.jit
def kernel(hidden_states, weight):
    wt = weight.T
    grid = (_TOKENS // _BT,)
    logits, wi_t = pl.pallas_call(
        _router_body,
        grid=grid,
        in_specs=[
            pl.BlockSpec((_BT, _HIDDEN), lambda i: (i, 0)),
            pl.BlockSpec((_HIDDEN, _E), lambda i: (0, 0)),
        ],
        out_specs=[
            pl.BlockSpec((_BT, _E), lambda i: (i, 0)),
            pl.BlockSpec((2 * _K, _BT), lambda i: (0, i)),
        ],
        out_shape=[
            jax.ShapeDtypeStruct((_TOKENS, _E), jnp.float32),
            jax.ShapeDtypeStruct((2 * _K, _TOKENS), jnp.float32),
        ],
        compiler_params=pltpu.CompilerParams(
            dimension_semantics=("arbitrary",),
        ),
    )(hidden_states, wt)
    wi = wi_t.T
    return (logits, wi[:, :_K], wi[:, _K:].astype(jnp.int32))
